# baseline (speedup 1.0000x reference)
import jax
import jax.numpy as jnp
from jax import lax
from jax.experimental import pallas as pl
from jax.experimental.pallas import tpu as pltpu
from jax.experimental.pallas import tpu_sc as plsc

B, V, M, L, C, D = 64, 8, 25, 32, 64, 128
S = V * B
CHUNK = 80
NCHUNK_W = 160
VCH = 64
NVCH_W = 16
NW = 32
SEG_PER_W = S // NW
B_PER_W = B // NW
IT_OFF = 0
VL_OFF = NCHUNK_W * CHUNK
V_OFF = 2 * VL_OFF
IDX_W = V_OFF + 3 * NVCH_W * VCH
OUT_ROWS = S + 3 * B


def _sc_body(idx_all, emb_i, emb_v, emb_c, emb_p, emb_d,
             out_hbm, ids, rows, ob,
             sa0, sb0, sa1, sb1, sa2, sb2, sa3, sb3):
    w = lax.axis_index("s") * 2 + lax.axis_index("c")
    zeros8 = tuple(jnp.zeros((16,), jnp.float32) for _ in range(8))
    zero = jnp.zeros((16,), jnp.float32)
    sems = ((sa0, sb0), (sa1, sb1), (sa2, sb2), (sa3, sb3))

    pltpu.sync_copy(idx_all.at[w], ids)

    def issue(t, k):
        sa, sb = sems[k]
        pltpu.async_copy(emb_i.at[ids.at[pl.ds(IT_OFF + t * CHUNK, CHUNK)]],
                         rows.at[pl.ds(160 * k, CHUNK)], sa)
        pltpu.async_copy(emb_v.at[ids.at[pl.ds(VL_OFF + t * CHUNK, CHUNK)]],
                         rows.at[pl.ds(160 * k + CHUNK, CHUNK)], sb)

    def wait_rows(base, n, sem):
        pltpu.make_async_copy(emb_i.at[pl.ds(0, n)],
                              rows.at[pl.ds(base, n)], sem).wait()

    def accum_pair(k, accs):
        ao, bo = 160 * k, 160 * k + CHUNK

        def row_body(r, a2):
            out = list(a2)
            for u in range(4):
                rr = r * 4 + u
                for j in range(8):
                    out[j] = out[j] + (rows[ao + rr, pl.ds(16 * j, 16)] *
                                       rows[bo + rr, pl.ds(16 * j, 16)])
            return tuple(out)

        return lax.fori_loop(0, CHUNK // 4, row_body, accs)

    for k in range(3):
        issue(k, k)

    def mon_body(i, accs):
        for u in range(4):
            t = 4 * i + u

            @pl.when(t + 3 < NCHUNK_W)
            def _(t=t, u=u):
                issue(t + 3, (u + 3) % 4)

            sa, sb = sems[u]
            wait_rows(160 * u, CHUNK, sa)
            wait_rows(160 * u + CHUNK, CHUNK, sb)
            accs = accum_pair(u, accs)
            flush = (t % 10) == 9

            @pl.when(flush)
            def _(t=t, accs=accs):
                sl = t // 10
                for j in range(8):
                    ob[sl, pl.ds(16 * j, 16)] = accs[j]

            accs = tuple(jnp.where(flush, zero, a) for a in accs)
        return accs

    lax.fori_loop(0, NCHUNK_W // 4, mon_body, zeros8)
    pltpu.sync_copy(ob.at[pl.ds(0, SEG_PER_W)],
                    out_hbm.at[pl.ds(w * SEG_PER_W, SEG_PER_W)])

    embs_v = (emb_c, emb_p, emb_d)

    def issue_v(t, u, odd):
        base = 160 * u + (CHUNK if odd else 0)
        sem = sems[u][1 if odd else 0]
        off = V_OFF + u * NVCH_W * VCH + t * VCH
        pltpu.async_copy(embs_v[u].at[ids.at[pl.ds(off, VCH)]],
                         rows.at[pl.ds(base, VCH)], sem)

    def accum_v(u, odd, accs):
        base = 160 * u + (CHUNK if odd else 0)

        def row_body(r, a2):
            out = list(a2)
            for q in range(4):
                rr = r * 4 + q
                for j in range(8):
                    out[j] = out[j] + rows[base + rr, pl.ds(16 * j, 16)]
            return tuple(out)

        return lax.fori_loop(0, VCH // 4, row_body, accs)

    for u in range(3):
        issue_v(0, u, False)
        issue_v(1, u, True)

    def vis_body(i, carry):
        accs3 = [list(carry[u * 8:(u + 1) * 8]) for u in range(3)]
        for u in range(3):
            for odd in (False, True):
                t = 2 * i + (1 if odd else 0)
                base = 160 * u + (CHUNK if odd else 0)
                wait_rows(base, VCH, sems[u][1 if odd else 0])
                accs3[u] = list(accum_v(u, odd, tuple(accs3[u])))

                @pl.when(t + 2 < NVCH_W)
                def _(t=t, u=u, odd=odd):
                    issue_v(t + 2, u, odd)

            flush = (i % 4) == 3

            @pl.when(flush)
            def _(i=i, u=u, a=accs3[u]):
                bl = i // 4
                for j in range(8):
                    ob[SEG_PER_W + u * B_PER_W + bl, pl.ds(16 * j, 16)] = a[j]

            accs3[u] = [jnp.where(flush, zero, a) for a in accs3[u]]
        return tuple(accs3[0] + accs3[1] + accs3[2])

    lax.fori_loop(0, NVCH_W // 2, vis_body,
                  tuple(jnp.zeros((16,), jnp.float32) for _ in range(24)))
    for u in range(3):
        pltpu.sync_copy(
            ob.at[pl.ds(SEG_PER_W + u * B_PER_W, B_PER_W)],
            out_hbm.at[pl.ds(S + u * B + w * B_PER_W, B_PER_W)])


_sc_pool = pl.kernel(
    _sc_body,
    out_type=jax.ShapeDtypeStruct((OUT_ROWS, D), jnp.float32),
    mesh=plsc.VectorSubcoreMesh(core_axis_name="c", subcore_axis_name="s"),
    scratch_types=[
        pltpu.VMEM((IDX_W,), jnp.int32),
        pltpu.VMEM((4 * 2 * CHUNK, D), jnp.float32),
        pltpu.VMEM((SEG_PER_W + 3 * B_PER_W, D), jnp.float32),
        pltpu.SemaphoreType.DMA,
        pltpu.SemaphoreType.DMA,
        pltpu.SemaphoreType.DMA,
        pltpu.SemaphoreType.DMA,
        pltpu.SemaphoreType.DMA,
        pltpu.SemaphoreType.DMA,
        pltpu.SemaphoreType.DMA,
        pltpu.SemaphoreType.DMA,
    ],
)


def _tc_body(big, weight, age,
             mon_W, mon_b, mlp_c_W, mlp_c_b, mlp_p_W, mlp_p_b,
             mlp_d_W, mlp_d_b, mlp_w_W, mlp_w_b, mlp_a_W, mlp_a_b,
             fc_w_W, fc_w_b, fc_a_W, fc_a_b, fcp_W, fcp_b, out):
    f32 = jnp.float32
    allb = big[...]
    pooled = lax.slice(allb, (0, 0), (S, D))
    sc_ = lax.slice(allb, (S, 0), (S + B, D))
    sp_ = lax.slice(allb, (S + B, 0), (S + 2 * B, D))
    sd_ = lax.slice(allb, (S + 2 * B, 0), (S + 3 * B, D))

    def mm(x, w_):
        return jnp.dot(x, w_[...], preferred_element_type=f32)

    h = jnp.maximum(mm(pooled, mon_W) + mon_b[...], 0.0)
    ri = lax.broadcasted_iota(jnp.int32, (B, S), 0)
    cj = lax.broadcasted_iota(jnp.int32, (B, S), 1)
    sm = (cj // V == ri).astype(f32)
    e0 = jnp.dot(sm, h, preferred_element_type=f32)

    e1 = jnp.maximum(mm(sc_, mlp_c_W) + mlp_c_b[...], 0.0)
    e2 = jnp.maximum(mm(sp_, mlp_p_W) + mlp_p_b[...], 0.0)
    e3 = jnp.maximum(mm(sd_, mlp_d_W) + mlp_d_b[...], 0.0)

    def scalar_feat(vals_ref, fcW, fcb, mlpW, mlpb):
        vals = vals_ref[...]
        nz = (vals != 0.0).astype(f32)
        s1 = jnp.sum(vals, axis=1, keepdims=True)
        n = jnp.sum(nz, axis=1, keepdims=True)
        hv = s1 * fcW[...] + n * fcb[...]
        return jnp.maximum(mm(hv, mlpW) + mlpb[...], 0.0)

    e4 = scalar_feat(weight, fc_w_W, fc_w_b, mlp_w_W, mlp_w_b)
    e5 = scalar_feat(age, fc_a_W, fc_a_b, mlp_a_W, mlp_a_b)

    acc = fcp_b[...]
    for i, e in enumerate((e0, e1, e2, e3, e4, e5)):
        acc = acc + jnp.dot(e, fcp_W[i * D:(i + 1) * D, :],
                            preferred_element_type=f32)
    out[...] = acc


def kernel(lab_item, lab_value, cond, proc, drug, weight, age,
           emb_lab_item, emb_lab_value, emb_cond, emb_proc, emb_drug,
           mon_W, mon_b,
           mlp_cond_W, mlp_cond_b, mlp_proc_W, mlp_proc_b, mlp_drug_W, mlp_drug_b,
           mlp_weight_W, mlp_weight_b, mlp_age_W, mlp_age_b,
           fc_weight_W, fc_weight_b, fc_age_W, fc_age_b,
           fc_patient_W, fc_patient_b):
    i32 = jnp.int32
    idx_all = jnp.concatenate(
        [lab_item.astype(i32).reshape(NW, VL_OFF),
         lab_value.astype(i32).reshape(NW, VL_OFF),
         cond.astype(i32).reshape(NW, NVCH_W * VCH),
         proc.astype(i32).reshape(NW, NVCH_W * VCH),
         drug.astype(i32).reshape(NW, NVCH_W * VCH)], axis=1)

    big = _sc_pool(idx_all, emb_lab_item, emb_lab_value,
                   emb_cond, emb_proc, emb_drug)

    r2 = lambda x: x.reshape(1, -1)
    out = pl.pallas_call(
        _tc_body,
        out_shape=jax.ShapeDtypeStruct((B, D), jnp.float32),
    )(big, weight, age,
      mon_W, r2(mon_b), mlp_cond_W, r2(mlp_cond_b), mlp_proc_W, r2(mlp_proc_b),
      mlp_drug_W, r2(mlp_drug_b), mlp_weight_W, r2(mlp_weight_b),
      mlp_age_W, r2(mlp_age_b),
      fc_weight_W, r2(fc_weight_b), fc_age_W, r2(fc_age_b),
      fc_patient_W, r2(fc_patient_b))
    return out
